# Initial kernel scaffold; baseline (speedup 1.0000x reference)
import jax
import jax.numpy as jnp
from jax.experimental import pallas as pl


def kernel(seq, item_emb):
    raise NotImplementedError("write your pallas kernel here")


# speedup vs baseline: 1.9572x; 1.9572x over previous
import functools

import jax
import jax.numpy as jnp
from jax import lax
from jax.experimental import pallas as pl
from jax.experimental.pallas import tpu as pltpu
from jax.experimental.pallas import tpu_sc as plsc

BATCH = 16384
SEQ_LEN = 200
D = 64
NUM_WORKERS = 32
ROWS_PER_WORKER = BATCH // NUM_WORKERS


def _rsqrt(nv):
    i = lax.bitcast_convert_type(nv, jnp.int32)
    y = lax.bitcast_convert_type(0x5F3759DF - (i >> 1), jnp.float32)
    half = nv * 0.5
    for _ in range(4):
        y = y * (1.5 - half * y * y)
    return y


def _body(seq_hbm, emb_hbm, out_hbm, idx_v, rows_v, out_v, sem):
    nc = 2
    wid = lax.axis_index("s") * nc + lax.axis_index("c")
    base = wid * ROWS_PER_WORKER

    def row_body(b, carry):
        pltpu.sync_copy(seq_hbm.at[base + b], idx_v)
        cp_a = pltpu.async_copy(
            emb_hbm.at[idx_v.at[pl.ds(0, 128)]], rows_v.at[pl.ds(0, 128)], sem
        )
        cp_b = pltpu.async_copy(
            emb_hbm.at[idx_v.at[pl.ds(128, 72)]], rows_v.at[pl.ds(128, 72)], sem
        )
        cp_a.wait()
        cp_b.wait()

        def red(l, acc):
            a0, a1, a2, a3 = acc
            return (
                a0 + rows_v[l, pl.ds(0, 16)],
                a1 + rows_v[l, pl.ds(16, 16)],
                a2 + rows_v[l, pl.ds(32, 16)],
                a3 + rows_v[l, pl.ds(48, 16)],
            )

        z = jnp.zeros((16,), jnp.float32)
        a0, a1, a2, a3 = lax.fori_loop(0, SEQ_LEN, red, (z, z, z, z))

        t = a0 * a0 + a1 * a1 + a2 * a2 + a3 * a3
        iota = lax.iota(jnp.int32, 16)
        for s in (8, 4, 2, 1):
            t = t + t.at[(iota + s) & 15].get(mode="promise_in_bounds")
        y = _rsqrt(jnp.maximum(t, 1e-24))
        out_v[b, pl.ds(0, 16)] = a0 * y
        out_v[b, pl.ds(16, 16)] = a1 * y
        out_v[b, pl.ds(32, 16)] = a2 * y
        out_v[b, pl.ds(48, 16)] = a3 * y
        return carry

    lax.fori_loop(0, ROWS_PER_WORKER, row_body, 0)
    pltpu.sync_copy(out_v, out_hbm.at[pl.ds(base, ROWS_PER_WORKER)])


@jax.jit
def kernel(seq, item_emb):
    mesh = plsc.VectorSubcoreMesh(core_axis_name="c", subcore_axis_name="s")
    f = pl.kernel(
        _body,
        out_type=jax.ShapeDtypeStruct((BATCH, D), jnp.float32),
        mesh=mesh,
        compiler_params=pltpu.CompilerParams(use_tc_tiling_on_sc=False),
        scratch_types=[
            pltpu.VMEM((SEQ_LEN,), jnp.int32),
            pltpu.VMEM((SEQ_LEN, D), jnp.float32),
            pltpu.VMEM((ROWS_PER_WORKER, D), jnp.float32),
            pltpu.SemaphoreType.DMA,
        ],
    )
    return f(seq, item_emb)


# speedup vs baseline: 3.7876x; 1.9352x over previous
import jax
import jax.numpy as jnp
from jax import lax
from jax.experimental import pallas as pl
from jax.experimental.pallas import tpu as pltpu
from jax.experimental.pallas import tpu_sc as plsc

BATCH = 16384
SEQ_LEN = 200
D = 64
NUM_WORKERS = 32
ROWS_PER_WORKER = BATCH // NUM_WORKERS
NBUF = 4
IGRP = 64
NGRP = ROWS_PER_WORKER // IGRP
STAGES = IGRP // NBUF
CHUNK0 = 128
CHUNK1 = SEQ_LEN - CHUNK0


def _rsqrt(nv):
    i = lax.bitcast_convert_type(nv, jnp.int32)
    y = lax.bitcast_convert_type(0x5F3759DF - (i >> 1), jnp.float32)
    half = nv * 0.5
    for _ in range(4):
        y = y * (1.5 - half * y * y)
    return y


def _body(seq_hbm, emb_hbm, out_hbm, idx_v, rows_v, out_v,
          sem_idx, sem0, sem1, sem2, sem3):
    sems = (sem0, sem1, sem2, sem3)
    nc = 2
    wid = lax.axis_index("s") * nc + lax.axis_index("c")
    base = wid * ROWS_PER_WORKER

    def fire_gather(cur, local_row, slot, sem):
        src = idx_v.at[cur, local_row]
        pltpu.async_copy(
            emb_hbm.at[src.at[pl.ds(0, CHUNK0)]],
            rows_v.at[slot, pl.ds(0, CHUNK0)], sem)
        pltpu.async_copy(
            emb_hbm.at[src.at[pl.ds(CHUNK0, CHUNK1)]],
            rows_v.at[slot, pl.ds(CHUNK0, CHUNK1)], sem)

    def wait_gather(slot, sem):
        pltpu.make_async_copy(
            emb_hbm.at[pl.ds(0, SEQ_LEN)], rows_v.at[slot], sem).wait()

    def process_slot(cur, gi, s, slot):
        wait_gather(slot, sems[slot])

        def red(l, acc):
            a0, a1, a2, a3 = acc
            for k in range(8):
                e = l * 8 + k
                a0 = a0 + rows_v[slot, e, pl.ds(0, 16)]
                a1 = a1 + rows_v[slot, e, pl.ds(16, 16)]
                a2 = a2 + rows_v[slot, e, pl.ds(32, 16)]
                a3 = a3 + rows_v[slot, e, pl.ds(48, 16)]
            return (a0, a1, a2, a3)

        z = jnp.zeros((16,), jnp.float32)
        a0, a1, a2, a3 = lax.fori_loop(0, SEQ_LEN // 8, red, (z, z, z, z))

        t = a0 * a0 + a1 * a1 + a2 * a2 + a3 * a3
        iota = lax.iota(jnp.int32, 16)
        for sh in (8, 4, 2, 1):
            t = t + t.at[(iota + sh) & 15].get(mode="promise_in_bounds")
        y = _rsqrt(jnp.maximum(t, 1e-24))
        row = gi * IGRP + s * NBUF + slot
        out_v[row, pl.ds(0, 16)] = a0 * y
        out_v[row, pl.ds(16, 16)] = a1 * y
        out_v[row, pl.ds(32, 16)] = a2 * y
        out_v[row, pl.ds(48, 16)] = a3 * y

    cp_idx = pltpu.async_copy(
        seq_hbm.at[pl.ds(base, IGRP)], idx_v.at[0], sem_idx)

    for gi in range(NGRP):
        cur = gi % 2
        cp_idx.wait()
        if gi + 1 < NGRP:
            cp_idx = pltpu.async_copy(
                seq_hbm.at[pl.ds(base + (gi + 1) * IGRP, IGRP)],
                idx_v.at[(gi + 1) % 2], sem_idx)

        for slot in range(NBUF):
            fire_gather(cur, slot, slot, sems[slot])

        def stage(s, _, cur=cur, gi=gi):
            for slot in range(NBUF):
                process_slot(cur, gi, s, slot)
                fire_gather(cur, s * NBUF + NBUF + slot, slot, sems[slot])
            return 0

        def last_stage(s, _, cur=cur, gi=gi):
            for slot in range(NBUF):
                process_slot(cur, gi, s, slot)
            return 0

        lax.fori_loop(0, STAGES - 1, stage, 0)
        last_stage(STAGES - 1, 0)

    pltpu.sync_copy(out_v, out_hbm.at[pl.ds(base, ROWS_PER_WORKER)])


@jax.jit
def kernel(seq, item_emb):
    mesh = plsc.VectorSubcoreMesh(core_axis_name="c", subcore_axis_name="s")
    f = pl.kernel(
        _body,
        out_type=jax.ShapeDtypeStruct((BATCH, D), jnp.float32),
        mesh=mesh,
        compiler_params=pltpu.CompilerParams(use_tc_tiling_on_sc=False),
        scratch_types=[
            pltpu.VMEM((2, IGRP, SEQ_LEN), jnp.int32),
            pltpu.VMEM((NBUF, SEQ_LEN, D), jnp.float32),
            pltpu.VMEM((ROWS_PER_WORKER, D), jnp.float32),
            pltpu.SemaphoreType.DMA,
            pltpu.SemaphoreType.DMA,
            pltpu.SemaphoreType.DMA,
            pltpu.SemaphoreType.DMA,
            pltpu.SemaphoreType.DMA,
        ],
    )
    return f(seq, item_emb)
